# baseline (speedup 1.0000x reference)
import functools

import jax
import jax.numpy as jnp
from jax import lax
from jax.experimental import pallas as pl
from jax.experimental.pallas import tpu as pltpu
from jax.experimental.pallas import tpu_sc as plsc

E = 64
H = 768
I = 1024
T = 2048
A2 = 2 * T
TILE = 128
AP = 4672
EPS = 1e-5
NW = 32
TPW = T // NW
GPW = A2 // NW


def _route_kernel(x_ref, nw_ref, gw_ref, xn_ref, p0_ref, p1_ref, wa_ref,
                  wb_ref, off_ref):
    x = x_ref[...]
    var = jnp.mean(x * x, axis=-1, keepdims=True)
    xn = x * lax.rsqrt(var + EPS) * nw_ref[...]
    xn_ref[...] = xn
    logits = lax.dot_general(xn, gw_ref[...], (((1,), (1,)), ((), ())),
                             preferred_element_type=jnp.float32)
    m = jnp.max(logits, axis=-1, keepdims=True)
    ex = jnp.exp(logits - m)
    p = ex / jnp.sum(ex, axis=-1, keepdims=True)
    col = lax.broadcasted_iota(jnp.int32, (T, E), 1)
    m1 = jnp.max(p, axis=-1, keepdims=True)
    i1 = jnp.min(jnp.where(p == m1, col, E), axis=-1, keepdims=True)
    pm = jnp.where(col == i1, -1.0, p)
    m2 = jnp.max(pm, axis=-1, keepdims=True)
    i2 = jnp.min(jnp.where(pm == m2, col, E), axis=-1, keepdims=True)
    wa_ref[...] = m1
    wb_ref[...] = m2
    a0 = (col == i1).astype(jnp.float32)
    a1 = (col == i2).astype(jnp.float32)
    c = a0 + a1
    ch = 256
    ri = lax.broadcasted_iota(jnp.int32, (ch, ch), 0)
    ci = lax.broadcasted_iota(jnp.int32, (ch, ch), 1)
    stril = (ci < ri).astype(jnp.float32)
    chunks = []
    prefix = jnp.zeros((1, E), jnp.float32)
    for k in range(T // ch):
        cc = lax.slice(c, (k * ch, 0), ((k + 1) * ch, E))
        chunks.append(
            lax.dot_general(stril, cc, (((1,), (0,)), ((), ())),
                            preferred_element_type=jnp.float32) + prefix)
        prefix = prefix + jnp.sum(cc, axis=0, keepdims=True)
    excl = jnp.concatenate(chunks, axis=0)
    cnt = prefix.astype(jnp.int32)
    pcnt = (((cnt + 7) // 8) * 8).astype(jnp.float32)
    er = lax.broadcasted_iota(jnp.int32, (E, E), 0)
    ec = lax.broadcasted_iota(jnp.int32, (E, E), 1)
    striu = (er < ec).astype(jnp.float32)
    poffs = lax.dot_general(pcnt, striu, (((1,), (0,)), ((), ())),
                            preferred_element_type=jnp.float32)
    off_ref[...] = jnp.concatenate([poffs, cnt.astype(jnp.float32)],
                                   axis=1).astype(jnp.int32)
    posbase = poffs + excl
    p0_ref[...] = jnp.sum(posbase * a0, axis=-1,
                          keepdims=True).astype(jnp.int32)
    p1_ref[...] = jnp.sum(posbase * a1, axis=-1,
                          keepdims=True).astype(jnp.int32)


_route = pl.pallas_call(
    _route_kernel,
    out_shape=[
        jax.ShapeDtypeStruct((T, H), jnp.float32),
        jax.ShapeDtypeStruct((T, 1), jnp.int32),
        jax.ShapeDtypeStruct((T, 1), jnp.int32),
        jax.ShapeDtypeStruct((T, 1), jnp.float32),
        jax.ShapeDtypeStruct((T, 1), jnp.float32),
        jax.ShapeDtypeStruct((1, 128), jnp.int32),
    ],
)


@functools.cache
def _sc_kernels():
    mesh = plsc.VectorSubcoreMesh(core_axis_name="c", subcore_axis_name="s")
    nc = mesh.num_cores

    @functools.partial(
        pl.kernel,
        mesh=mesh,
        out_type=jax.ShapeDtypeStruct((AP, H), jnp.float32),
        scratch_types=[
            pltpu.VMEM((TPW, H), jnp.float32),
            pltpu.VMEM((TPW,), jnp.int32),
            pltpu.VMEM((TPW,), jnp.int32),
            pltpu.SemaphoreType.DMA,
            pltpu.SemaphoreType.DMA,
        ],
    )
    def _sc_dispatch(xn_hbm, p0_hbm, p1_hbm, xs_hbm, rows_v, idx0_v, idx1_v,
                     sem0, sem1):
        wid = lax.axis_index("s") * nc + lax.axis_index("c")
        base = wid * TPW
        pltpu.sync_copy(p0_hbm.at[pl.ds(base, TPW)], idx0_v)
        pltpu.sync_copy(p1_hbm.at[pl.ds(base, TPW)], idx1_v)
        pltpu.sync_copy(xn_hbm.at[pl.ds(base, TPW), :], rows_v)
        c0 = pltpu.async_copy(rows_v, xs_hbm.at[idx0_v], sem0)
        c1 = pltpu.async_copy(rows_v, xs_hbm.at[idx1_v], sem1)
        c0.wait()
        c1.wait()

    @functools.partial(
        pl.kernel,
        mesh=mesh,
        out_type=jax.ShapeDtypeStruct((A2, H), jnp.float32),
        scratch_types=[
            pltpu.VMEM((GPW // 2, H), jnp.float32),
            pltpu.VMEM((GPW // 2, H), jnp.float32),
            pltpu.VMEM((GPW // 2,), jnp.int32),
            pltpu.VMEM((GPW // 2,), jnp.int32),
            pltpu.SemaphoreType.DMA,
            pltpu.SemaphoreType.DMA,
        ],
    )
    def _sc_combine(ys_hbm, pos_hbm, yg_hbm, rows0_v, rows1_v, idx0_v,
                    idx1_v, sem0, sem1):
        wid = lax.axis_index("s") * nc + lax.axis_index("c")
        hw = GPW // 2
        base = wid * GPW
        pltpu.sync_copy(pos_hbm.at[pl.ds(base, hw)], idx0_v)
        c0 = pltpu.async_copy(ys_hbm.at[idx0_v], rows0_v, sem0)
        pltpu.sync_copy(pos_hbm.at[pl.ds(base + hw, hw)], idx1_v)
        c1 = pltpu.async_copy(ys_hbm.at[idx1_v], rows1_v, sem1)
        c0.wait()
        pltpu.sync_copy(rows0_v, yg_hbm.at[pl.ds(base, hw), :])
        c1.wait()
        pltpu.sync_copy(rows1_v, yg_hbm.at[pl.ds(base + hw, hw), :])

    return _sc_dispatch, _sc_combine


def _moe_kernel(off_ref, xs_ref, ws_ref, w2s_ref, ys_ref):
    e = pl.program_id(0)
    start = off_ref[e]
    n = off_ref[E + e]
    ntiles = (n + TILE - 1) // TILE
    w1 = ws_ref[0, :I, :]
    w3 = ws_ref[0, I:, :]
    w2 = w2s_ref[0]

    def body(i, carry):
        s = pl.multiple_of(start + i * TILE, 8)
        xt = xs_ref[pl.ds(s, TILE), :]
        g = lax.dot_general(xt, w1, (((1,), (1,)), ((), ())),
                            preferred_element_type=jnp.float32)
        u = lax.dot_general(xt, w3, (((1,), (1,)), ((), ())),
                            preferred_element_type=jnp.float32)
        h = g * (1.0 / (1.0 + jnp.exp(-g))) * u
        ye = lax.dot_general(h, w2, (((1,), (1,)), ((), ())),
                             preferred_element_type=jnp.float32)
        ys_ref[pl.ds(s, TILE), :] = ye
        return carry

    lax.fori_loop(0, ntiles, body, 0)


_moe = pl.pallas_call(
    _moe_kernel,
    grid_spec=pltpu.PrefetchScalarGridSpec(
        num_scalar_prefetch=1,
        grid=(E,),
        in_specs=[
            pl.BlockSpec((AP, H), lambda e, off: (0, 0)),
            pl.BlockSpec((1, 2 * I, H), lambda e, off: (e, 0, 0)),
            pl.BlockSpec((1, H, I), lambda e, off: (e, 0, 0)),
        ],
        out_specs=pl.BlockSpec((AP, H), lambda e, off: (0, 0)),
    ),
    out_shape=jax.ShapeDtypeStruct((AP, H), jnp.float32),
)


def _combine_kernel(r_ref, y0_ref, y1_ref, wa_ref, wb_ref, o_ref):
    o_ref[...] = (r_ref[...] + wa_ref[...] * y0_ref[...]
                  + wb_ref[...] * y1_ref[...])


_combine = pl.pallas_call(
    _combine_kernel,
    grid=(8,),
    in_specs=[
        pl.BlockSpec((T // 8, H), lambda b: (b, 0)),
        pl.BlockSpec((T // 8, H), lambda b: (b, 0)),
        pl.BlockSpec((T // 8, H), lambda b: (b + 8, 0)),
        pl.BlockSpec((T // 8, 1), lambda b: (b, 0)),
        pl.BlockSpec((T // 8, 1), lambda b: (b, 0)),
    ],
    out_specs=pl.BlockSpec((T // 8, H), lambda b: (b, 0)),
    out_shape=jax.ShapeDtypeStruct((T, H), jnp.float32),
)


def kernel(hidden_states, norm_weight, gate_w, ws, w2s):
    B, S, _ = hidden_states.shape
    x2 = hidden_states.reshape(T, H)
    xn, p0, p1, wa, wb, off = _route(x2, norm_weight.reshape(1, H), gate_w)
    p0f = p0.reshape(T)
    p1f = p1.reshape(T)
    sc_dispatch, sc_combine = _sc_kernels()
    xs = sc_dispatch(xn, p0f, p1f)
    ys = _moe(off.reshape(128), xs, ws, w2s)
    yg = sc_combine(ys, jnp.concatenate([p0f, p1f], axis=0))
    out = _combine(x2, yg, yg, wa, wb)
    return out.reshape(B, S, H)


# speedup vs baseline: 1.0029x; 1.0029x over previous
import functools

import jax
import jax.numpy as jnp
from jax import lax
from jax.experimental import pallas as pl
from jax.experimental.pallas import tpu as pltpu
from jax.experimental.pallas import tpu_sc as plsc

E = 64
H = 768
I = 1024
T = 2048
A2 = 2 * T
TILE = 128
AP = 4672
EPS = 1e-5
NW = 32
TPW = T // NW
GPW = A2 // NW


def _route_kernel(x_ref, nw_ref, gw_ref, xn_ref, p0_ref, p1_ref, wa_ref,
                  wb_ref, off_ref):
    x = x_ref[...]
    var = jnp.mean(x * x, axis=-1, keepdims=True)
    xn = x * lax.rsqrt(var + EPS) * nw_ref[...]
    xn_ref[...] = xn
    logits = lax.dot_general(xn, gw_ref[...], (((1,), (1,)), ((), ())),
                             preferred_element_type=jnp.float32)
    m = jnp.max(logits, axis=-1, keepdims=True)
    ex = jnp.exp(logits - m)
    p = ex / jnp.sum(ex, axis=-1, keepdims=True)
    col = lax.broadcasted_iota(jnp.int32, (T, E), 1)
    m1 = jnp.max(p, axis=-1, keepdims=True)
    i1 = jnp.min(jnp.where(p == m1, col, E), axis=-1, keepdims=True)
    pm = jnp.where(col == i1, -1.0, p)
    m2 = jnp.max(pm, axis=-1, keepdims=True)
    i2 = jnp.min(jnp.where(pm == m2, col, E), axis=-1, keepdims=True)
    wa_ref[...] = m1
    wb_ref[...] = m2
    a0 = (col == i1).astype(jnp.float32)
    a1 = (col == i2).astype(jnp.float32)
    c = a0 + a1
    ch = 256
    ri = lax.broadcasted_iota(jnp.int32, (ch, ch), 0)
    ci = lax.broadcasted_iota(jnp.int32, (ch, ch), 1)
    stril = (ci < ri).astype(jnp.float32)
    chunks = []
    prefix = jnp.zeros((1, E), jnp.float32)
    for k in range(T // ch):
        cc = lax.slice(c, (k * ch, 0), ((k + 1) * ch, E))
        chunks.append(
            lax.dot_general(stril, cc, (((1,), (0,)), ((), ())),
                            preferred_element_type=jnp.float32) + prefix)
        prefix = prefix + jnp.sum(cc, axis=0, keepdims=True)
    excl = jnp.concatenate(chunks, axis=0)
    cnt = prefix.astype(jnp.int32)
    pcnt = (((cnt + 7) // 8) * 8).astype(jnp.float32)
    er = lax.broadcasted_iota(jnp.int32, (E, E), 0)
    ec = lax.broadcasted_iota(jnp.int32, (E, E), 1)
    striu = (er < ec).astype(jnp.float32)
    poffs = lax.dot_general(pcnt, striu, (((1,), (0,)), ((), ())),
                            preferred_element_type=jnp.float32)
    off_ref[...] = jnp.concatenate([poffs, cnt.astype(jnp.float32)],
                                   axis=1).astype(jnp.int32)
    posbase = poffs + excl
    p0_ref[...] = jnp.sum(posbase * a0, axis=-1,
                          keepdims=True).astype(jnp.int32)
    p1_ref[...] = jnp.sum(posbase * a1, axis=-1,
                          keepdims=True).astype(jnp.int32)


_route = pl.pallas_call(
    _route_kernel,
    out_shape=[
        jax.ShapeDtypeStruct((T, H), jnp.float32),
        jax.ShapeDtypeStruct((T, 1), jnp.int32),
        jax.ShapeDtypeStruct((T, 1), jnp.int32),
        jax.ShapeDtypeStruct((T, 1), jnp.float32),
        jax.ShapeDtypeStruct((T, 1), jnp.float32),
        jax.ShapeDtypeStruct((1, 128), jnp.int32),
    ],
)


@functools.cache
def _sc_kernels():
    mesh = plsc.VectorSubcoreMesh(core_axis_name="c", subcore_axis_name="s")
    nc = mesh.num_cores

    @functools.partial(
        pl.kernel,
        mesh=mesh,
        out_type=jax.ShapeDtypeStruct((AP, H), jnp.float32),
        scratch_types=[
            pltpu.VMEM((TPW, H), jnp.float32),
            pltpu.VMEM((TPW,), jnp.int32),
            pltpu.VMEM((TPW,), jnp.int32),
            pltpu.SemaphoreType.DMA,
            pltpu.SemaphoreType.DMA,
        ],
    )
    def _sc_dispatch(xn_hbm, p0_hbm, p1_hbm, xs_hbm, rows_v, idx0_v, idx1_v,
                     sem0, sem1):
        wid = lax.axis_index("s") * nc + lax.axis_index("c")
        base = wid * TPW
        pltpu.sync_copy(p0_hbm.at[pl.ds(base, TPW)], idx0_v)
        pltpu.sync_copy(p1_hbm.at[pl.ds(base, TPW)], idx1_v)
        pltpu.sync_copy(xn_hbm.at[pl.ds(base, TPW), :], rows_v)
        c0 = pltpu.async_copy(rows_v, xs_hbm.at[idx0_v], sem0)
        c1 = pltpu.async_copy(rows_v, xs_hbm.at[idx1_v], sem1)
        c0.wait()
        c1.wait()

    @functools.partial(
        pl.kernel,
        mesh=mesh,
        out_type=jax.ShapeDtypeStruct((A2, H), jnp.float32),
        scratch_types=[
            pltpu.VMEM((GPW // 2, H), jnp.float32),
            pltpu.VMEM((GPW // 2, H), jnp.float32),
            pltpu.VMEM((GPW // 2,), jnp.int32),
            pltpu.VMEM((GPW // 2,), jnp.int32),
            pltpu.SemaphoreType.DMA,
            pltpu.SemaphoreType.DMA,
        ],
    )
    def _sc_combine(ys_hbm, pos_hbm, yg_hbm, rows0_v, rows1_v, idx0_v,
                    idx1_v, sem0, sem1):
        wid = lax.axis_index("s") * nc + lax.axis_index("c")
        hw = GPW // 2
        base = wid * GPW
        pltpu.sync_copy(pos_hbm.at[pl.ds(base, hw)], idx0_v)
        c0 = pltpu.async_copy(ys_hbm.at[idx0_v], rows0_v, sem0)
        pltpu.sync_copy(pos_hbm.at[pl.ds(base + hw, hw)], idx1_v)
        c1 = pltpu.async_copy(ys_hbm.at[idx1_v], rows1_v, sem1)
        c0.wait()
        pltpu.sync_copy(rows0_v, yg_hbm.at[pl.ds(base, hw), :])
        c1.wait()
        pltpu.sync_copy(rows1_v, yg_hbm.at[pl.ds(base + hw, hw), :])

    return _sc_dispatch, _sc_combine


def _moe_kernel(off_ref, xs_ref, w1_ref, w3_ref, w2s_ref, ys_ref):
    e = pl.program_id(0)
    start = off_ref[e]
    n = off_ref[E + e]
    ntiles = (n + TILE - 1) // TILE
    w1 = w1_ref[0, 0]
    w3 = w3_ref[0, 0]
    w2 = w2s_ref[0]

    def body(i, carry):
        s = pl.multiple_of(start + i * TILE, 8)
        xt = xs_ref[pl.ds(s, TILE), :]
        g = lax.dot_general(xt, w1, (((1,), (1,)), ((), ())),
                            preferred_element_type=jnp.float32)
        u = lax.dot_general(xt, w3, (((1,), (1,)), ((), ())),
                            preferred_element_type=jnp.float32)
        h = g * (1.0 / (1.0 + jnp.exp(-g))) * u
        ye = lax.dot_general(h, w2, (((1,), (1,)), ((), ())),
                             preferred_element_type=jnp.float32)
        ys_ref[pl.ds(s, TILE), :] = ye
        return carry

    lax.fori_loop(0, ntiles, body, 0)


_moe = pl.pallas_call(
    _moe_kernel,
    grid_spec=pltpu.PrefetchScalarGridSpec(
        num_scalar_prefetch=1,
        grid=(E,),
        in_specs=[
            pl.BlockSpec((AP, H), lambda e, off: (0, 0)),
            pl.BlockSpec((1, 1, I, H), lambda e, off: (e, 0, 0, 0)),
            pl.BlockSpec((1, 1, I, H), lambda e, off: (e, 1, 0, 0)),
            pl.BlockSpec((1, H, I), lambda e, off: (e, 0, 0)),
        ],
        out_specs=pl.BlockSpec((AP, H), lambda e, off: (0, 0)),
    ),
    out_shape=jax.ShapeDtypeStruct((AP, H), jnp.float32),
)


def _combine_kernel(r_ref, y0_ref, y1_ref, wa_ref, wb_ref, o_ref):
    o_ref[...] = (r_ref[...] + wa_ref[...] * y0_ref[...]
                  + wb_ref[...] * y1_ref[...])


_combine = pl.pallas_call(
    _combine_kernel,
    grid=(8,),
    in_specs=[
        pl.BlockSpec((T // 8, H), lambda b: (b, 0)),
        pl.BlockSpec((T // 8, H), lambda b: (b, 0)),
        pl.BlockSpec((T // 8, H), lambda b: (b + 8, 0)),
        pl.BlockSpec((T // 8, 1), lambda b: (b, 0)),
        pl.BlockSpec((T // 8, 1), lambda b: (b, 0)),
    ],
    out_specs=pl.BlockSpec((T // 8, H), lambda b: (b, 0)),
    out_shape=jax.ShapeDtypeStruct((T, H), jnp.float32),
)


def kernel(hidden_states, norm_weight, gate_w, ws, w2s):
    B, S, _ = hidden_states.shape
    x2 = hidden_states.reshape(T, H)
    xn, p0, p1, wa, wb, off = _route(x2, norm_weight.reshape(1, H), gate_w)
    p0f = p0.reshape(T)
    p1f = p1.reshape(T)
    sc_dispatch, sc_combine = _sc_kernels()
    xs = sc_dispatch(xn, p0f, p1f)
    ws4 = ws.reshape(E, 2, I, H)
    ys = _moe(off.reshape(128), xs, ws4, ws4, w2s)
    yg = sc_combine(ys, jnp.concatenate([p0f, p1f], axis=0))
    out = _combine(x2, yg, yg, wa, wb)
    return out.reshape(B, S, H)
